# Initial kernel scaffold; baseline (speedup 1.0000x reference)
import jax
import jax.numpy as jnp
from jax.experimental import pallas as pl


def kernel(x, src, tgt, Mtgt, W1, b1, W2, b2):
    raise NotImplementedError("write your pallas kernel here")


# speedup vs baseline: 6.8578x; 6.8578x over previous
import functools

import jax
import jax.numpy as jnp
from jax import lax
from jax.experimental import pallas as pl
from jax.experimental.pallas import tpu as pltpu
from jax.experimental.pallas import tpu_sc as plsc

N_NODES = 10000
N_EDGES = 320000
NFEAT = 128
NHID = 128
NCLASS = 40
CPAD = 48

K = 125
NCHUNK = 80
STRIPE = 640


def _make_sc_agg(F):
    mesh = plsc.VectorSubcoreMesh(core_axis_name="c", subcore_axis_name="s")

    @functools.partial(
        pl.kernel,
        out_type=jax.ShapeDtypeStruct((2 * N_NODES, F), jnp.float32),
        mesh=mesh,
        compiler_params=pltpu.CompilerParams(
            needs_layout_passes=False, use_tc_tiling_on_sc=False
        ),
        scratch_types=[
            pltpu.VMEM((NCHUNK, K), jnp.int32),
            pltpu.VMEM((NCHUNK, K), jnp.int32),
            pltpu.VMEM((NCHUNK * K,), jnp.float32),
            pltpu.VMEM((K, F), jnp.float32),
            pltpu.VMEM_SHARED((N_NODES, F), jnp.float32),
            pltpu.SemaphoreType.DMA,
        ],
    )
    def sc_agg(h_hbm, srcr_hbm, tgtr_hbm, m_hbm, out_hbm,
               srcb, tgtb, mb, rows, accum, sem):
        c = lax.axis_index("c")
        s = lax.axis_index("s")
        w = c * 16 + s

        pltpu.sync_copy(srcr_hbm.at[pl.ds(w * NCHUNK, NCHUNK)], srcb)
        pltpu.sync_copy(tgtr_hbm.at[pl.ds(w * NCHUNK, NCHUNK)], tgtb)
        pltpu.sync_copy(m_hbm.at[pl.ds(w * (NCHUNK * K), NCHUNK * K)], mb)

        def zrow(r, _):
            for j in range(F // 16):
                rows[r, pl.ds(j * 16, 16)] = jnp.zeros((16,), jnp.float32)
            return 0
        lax.fori_loop(0, K, zrow, 0)
        for kk in range(STRIPE // 80):
            start = jnp.minimum(s * STRIPE + kk * 80, N_NODES - 80)
            pltpu.sync_copy(rows.at[pl.ds(0, 80)], accum.at[pl.ds(start, 80)])
        plsc.subcore_barrier()

        lane0 = jnp.zeros((16,), jnp.int32)

        def chunk(i, _):
            pltpu.async_copy(h_hbm.at[srcb.at[i]], rows, sem).wait()

            def edge(e, _):
                mv = plsc.load_gather(mb, [lane0 + (i * K + e)])
                for j in range(F // 16):
                    sl = pl.ds(j * 16, 16)
                    rows[e, sl] = rows[e, sl] * mv
                return 0
            lax.fori_loop(0, K, edge, 0)

            pltpu.sync_copy(rows, accum.at[tgtb.at[i]], add=True)
            return 0
        lax.fori_loop(0, NCHUNK, chunk, 0)

        plsc.subcore_barrier()
        ostart = jnp.minimum(s * STRIPE, N_NODES - STRIPE)
        pltpu.sync_copy(
            accum.at[pl.ds(ostart, STRIPE)],
            out_hbm.at[pl.ds(c * N_NODES + ostart, STRIPE)],
        )

    return sc_agg


_sc_agg_h = _make_sc_agg(NHID)
_sc_agg_c = _make_sc_agg(CPAD)

_BR = 1000
_G = N_NODES // _BR


def _mm1_body(x_ref, w_ref, o_ref):
    o_ref[...] = jnp.dot(x_ref[...], w_ref[...], preferred_element_type=jnp.float32)


def _mm1(x, W1):
    return pl.pallas_call(
        _mm1_body,
        grid=(_G,),
        in_specs=[
            pl.BlockSpec((_BR, NFEAT), lambda i: (i, 0)),
            pl.BlockSpec((NFEAT, NHID), lambda i: (0, 0)),
        ],
        out_specs=pl.BlockSpec((_BR, NHID), lambda i: (i, 0)),
        out_shape=jax.ShapeDtypeStruct((N_NODES, NHID), jnp.float32),
    )(x, W1)


def _mm2_body(p0_ref, p1_ref, b1_ref, w_ref, o_ref):
    h = jnp.maximum(p0_ref[...] + p1_ref[...] + b1_ref[...], 0.0)
    o_ref[...] = jnp.dot(h, w_ref[...], preferred_element_type=jnp.float32)


def _mm2(p, b1, W2p):
    return pl.pallas_call(
        _mm2_body,
        grid=(_G,),
        in_specs=[
            pl.BlockSpec((_BR, NHID), lambda i: (i, 0)),
            pl.BlockSpec((_BR, NHID), lambda i: (i + _G, 0)),
            pl.BlockSpec((1, NHID), lambda i: (0, 0)),
            pl.BlockSpec((NHID, CPAD), lambda i: (0, 0)),
        ],
        out_specs=pl.BlockSpec((_BR, CPAD), lambda i: (i, 0)),
        out_shape=jax.ShapeDtypeStruct((N_NODES, CPAD), jnp.float32),
    )(p, p, b1, W2p)


def _fin_body(q0_ref, q1_ref, b2_ref, o_ref):
    z = q0_ref[...] + q1_ref[...] + b2_ref[...]
    col = lax.broadcasted_iota(jnp.int32, z.shape, 1)
    zm = jnp.where(col < NCLASS, z, -jnp.inf)
    m = jnp.max(zm, axis=1, keepdims=True)
    ls = jnp.log(jnp.sum(jnp.exp(zm - m), axis=1, keepdims=True))
    o_ref[...] = (z - m - ls)[:, :NCLASS]


def _fin(q, b2p):
    return pl.pallas_call(
        _fin_body,
        grid=(_G,),
        in_specs=[
            pl.BlockSpec((_BR, CPAD), lambda i: (i, 0)),
            pl.BlockSpec((_BR, CPAD), lambda i: (i + _G, 0)),
            pl.BlockSpec((1, CPAD), lambda i: (0, 0)),
        ],
        out_specs=pl.BlockSpec((_BR, NCLASS), lambda i: (i, 0)),
        out_shape=jax.ShapeDtypeStruct((N_NODES, NCLASS), jnp.float32),
    )(q, q, b2p)


def kernel(x, src, tgt, Mtgt, W1, b1, W2, b2):
    src = src.astype(jnp.int32).reshape(32 * NCHUNK, K)
    tgt = tgt.astype(jnp.int32).reshape(32 * NCHUNK, K)
    h1 = _mm1(x, W1)
    p = _sc_agg_h(h1, src, tgt, Mtgt)
    W2p = jnp.pad(W2, ((0, 0), (0, CPAD - NCLASS)))
    b2p = jnp.pad(b2, (0, CPAD - NCLASS)).reshape(1, CPAD)
    h2 = _mm2(p, b1.reshape(1, NHID), W2p)
    q = _sc_agg_c(h2, src, tgt, Mtgt)
    return _fin(q, b2p)


# speedup vs baseline: 10.8803x; 1.5866x over previous
import functools

import jax
import jax.numpy as jnp
from jax import lax
from jax.experimental import pallas as pl
from jax.experimental.pallas import tpu as pltpu
from jax.experimental.pallas import tpu_sc as plsc

N_NODES = 10000
N_EDGES = 320000
NFEAT = 128
NHID = 128
NCLASS = 40
CPAD = 48

K = 125
NCHUNK = 80
STRIPE = 640


def _make_sc_agg(F):
    mesh = plsc.VectorSubcoreMesh(core_axis_name="c", subcore_axis_name="s")

    @functools.partial(
        pl.kernel,
        out_type=jax.ShapeDtypeStruct((2 * N_NODES, F), jnp.float32),
        mesh=mesh,
        compiler_params=pltpu.CompilerParams(
            needs_layout_passes=False, use_tc_tiling_on_sc=False
        ),
        scratch_types=[
            pltpu.VMEM((NCHUNK, K), jnp.int32),
            pltpu.VMEM((K,), jnp.int32),
            pltpu.VMEM((K,), jnp.int32),
            pltpu.VMEM((K,), jnp.float32),
            pltpu.VMEM((K,), jnp.float32),
            pltpu.VMEM((K, F), jnp.float32),
            pltpu.VMEM((K, F), jnp.float32),
            pltpu.VMEM_SHARED((N_NODES, F), jnp.float32),
            pltpu.SemaphoreType.DMA,
            pltpu.SemaphoreType.DMA,
            pltpu.SemaphoreType.DMA,
            pltpu.SemaphoreType.DMA,
            pltpu.SemaphoreType.DMA,
            pltpu.SemaphoreType.DMA,
        ],
    )
    def sc_agg(h_hbm, srcr_hbm, tgtr_hbm, m_hbm, out_hbm,
               srcb, tgt0, tgt1, mb0, mb1, rows0, rows1, accum,
               semg0, semg1, semt0, semt1, semm0, semm1):
        c = lax.axis_index("c")
        s = lax.axis_index("s")
        w = c * 16 + s
        rowsb = (rows0, rows1)
        tgts = (tgt0, tgt1)
        mbs = (mb0, mb1)
        semg = (semg0, semg1)
        semt = (semt0, semt1)
        semm = (semm0, semm1)

        pltpu.sync_copy(srcr_hbm.at[pl.ds(w * NCHUNK, NCHUNK)], srcb)
        pltpu.sync_copy(tgtr_hbm.at[w * NCHUNK], tgt0)
        pltpu.sync_copy(m_hbm.at[w * NCHUNK], mb0)

        def zrow(r, _):
            for j in range(F // 16):
                rows0[r, pl.ds(j * 16, 16)] = jnp.zeros((16,), jnp.float32)
            return 0
        lax.fori_loop(0, K, zrow, 0)
        for kk in range(STRIPE // 80):
            start = jnp.minimum(s * STRIPE + kk * 80, N_NODES - 80)
            pltpu.sync_copy(rows0.at[pl.ds(0, 80)], accum.at[pl.ds(start, 80)])
        plsc.subcore_barrier()

        lane0 = jnp.zeros((16,), jnp.int32)

        pltpu.make_async_copy(h_hbm.at[srcb.at[0]], rows0, semg0).start()
        pltpu.make_async_copy(tgtr_hbm.at[w * NCHUNK + 1], tgt1, semt1).start()
        pltpu.make_async_copy(m_hbm.at[w * NCHUNK + 1], mb1, semm1).start()

        def outer(o, _):
            for b in range(2):
                i = o * 2 + b
                nb = 1 - b
                cur = rowsb[b]
                pltpu.make_async_copy(h_hbm.at[srcb.at[i]], cur, semg[b]).wait()

                @pl.when(i + 1 < NCHUNK)
                def _():
                    pltpu.make_async_copy(
                        tgtr_hbm.at[w * NCHUNK + i + 1], tgts[nb], semt[nb]
                    ).wait()
                    pltpu.make_async_copy(
                        m_hbm.at[w * NCHUNK + i + 1], mbs[nb], semm[nb]
                    ).wait()
                    pltpu.make_async_copy(
                        h_hbm.at[srcb.at[i + 1]], rowsb[nb], semg[nb]
                    ).start()

                @plsc.parallel_loop(0, K, unroll=5)
                def edge(e):
                    mv = plsc.load_gather(mbs[b], [lane0 + e])
                    for j in range(F // 16):
                        sl = pl.ds(j * 16, 16)
                        cur[e, sl] = cur[e, sl] * mv

                pltpu.sync_copy(cur, accum.at[tgts[b]], add=True)

                @pl.when(i + 2 < NCHUNK)
                def _():
                    pltpu.make_async_copy(
                        tgtr_hbm.at[w * NCHUNK + i + 2], tgts[b], semt[b]
                    ).start()
                    pltpu.make_async_copy(
                        m_hbm.at[w * NCHUNK + i + 2], mbs[b], semm[b]
                    ).start()
            return 0
        lax.fori_loop(0, NCHUNK // 2, outer, 0)

        plsc.subcore_barrier()
        ostart = jnp.minimum(s * STRIPE, N_NODES - STRIPE)
        pltpu.sync_copy(
            accum.at[pl.ds(ostart, STRIPE)],
            out_hbm.at[pl.ds(c * N_NODES + ostart, STRIPE)],
        )

    return sc_agg


_sc_agg_h = _make_sc_agg(NHID)
_sc_agg_c = _make_sc_agg(CPAD)

_BR = 1000
_G = N_NODES // _BR


def _mm1_body(x_ref, w_ref, o_ref):
    o_ref[...] = jnp.dot(x_ref[...], w_ref[...], preferred_element_type=jnp.float32)


def _mm1(x, W1):
    return pl.pallas_call(
        _mm1_body,
        grid=(_G,),
        in_specs=[
            pl.BlockSpec((_BR, NFEAT), lambda i: (i, 0)),
            pl.BlockSpec((NFEAT, NHID), lambda i: (0, 0)),
        ],
        out_specs=pl.BlockSpec((_BR, NHID), lambda i: (i, 0)),
        out_shape=jax.ShapeDtypeStruct((N_NODES, NHID), jnp.float32),
    )(x, W1)


def _mm2_body(p0_ref, p1_ref, b1_ref, w_ref, o_ref):
    h = jnp.maximum(p0_ref[...] + p1_ref[...] + b1_ref[...], 0.0)
    o_ref[...] = jnp.dot(h, w_ref[...], preferred_element_type=jnp.float32)


def _mm2(p, b1, W2p):
    return pl.pallas_call(
        _mm2_body,
        grid=(_G,),
        in_specs=[
            pl.BlockSpec((_BR, NHID), lambda i: (i, 0)),
            pl.BlockSpec((_BR, NHID), lambda i: (i + _G, 0)),
            pl.BlockSpec((1, NHID), lambda i: (0, 0)),
            pl.BlockSpec((NHID, CPAD), lambda i: (0, 0)),
        ],
        out_specs=pl.BlockSpec((_BR, CPAD), lambda i: (i, 0)),
        out_shape=jax.ShapeDtypeStruct((N_NODES, CPAD), jnp.float32),
    )(p, p, b1, W2p)


def _fin_body(q0_ref, q1_ref, b2_ref, o_ref):
    z = q0_ref[...] + q1_ref[...] + b2_ref[...]
    col = lax.broadcasted_iota(jnp.int32, z.shape, 1)
    zm = jnp.where(col < NCLASS, z, -jnp.inf)
    m = jnp.max(zm, axis=1, keepdims=True)
    ls = jnp.log(jnp.sum(jnp.exp(zm - m), axis=1, keepdims=True))
    o_ref[...] = (z - m - ls)[:, :NCLASS]


def _fin(q, b2p):
    return pl.pallas_call(
        _fin_body,
        grid=(_G,),
        in_specs=[
            pl.BlockSpec((_BR, CPAD), lambda i: (i, 0)),
            pl.BlockSpec((_BR, CPAD), lambda i: (i + _G, 0)),
            pl.BlockSpec((1, CPAD), lambda i: (0, 0)),
        ],
        out_specs=pl.BlockSpec((_BR, NCLASS), lambda i: (i, 0)),
        out_shape=jax.ShapeDtypeStruct((N_NODES, NCLASS), jnp.float32),
    )(q, q, b2p)


def kernel(x, src, tgt, Mtgt, W1, b1, W2, b2):
    src = src.astype(jnp.int32).reshape(32 * NCHUNK, K)
    tgt = tgt.astype(jnp.int32).reshape(32 * NCHUNK, K)
    Mtgt = Mtgt.reshape(32 * NCHUNK, K)
    h1 = _mm1(x, W1)
    p = _sc_agg_h(h1, src, tgt, Mtgt)
    W2p = jnp.pad(W2, ((0, 0), (0, CPAD - NCLASS)))
    b2p = jnp.pad(b2, (0, CPAD - NCLASS)).reshape(1, CPAD)
    h2 = _mm2(p, b1.reshape(1, NHID), W2p)
    q = _sc_agg_c(h2, src, tgt, Mtgt)
    return _fin(q, b2p)
